# baseline (speedup 1.0000x reference)
import jax
import jax.numpy as jnp
import numpy as np
from jax.experimental import pallas as pl
from jax.experimental.pallas import tpu as pltpu

BN = 512


def _fused_kernel(xt_ref, w1_ref, w2_ref, fc1_ref, f2_ref, f3_ref, bp_ref,
                  bias_ref, out_ref, c1_ref, p1_ref, c2_ref, p2_ref):
    f32 = jnp.float32
    bf16 = jnp.bfloat16

    for i in range(24):
        c1_ref[i] = jnp.dot(w1_ref[...],
                            xt_ref[pl.ds(28 * i, 140), :].astype(bf16),
                            preferred_element_type=f32)

    b1 = bp_ref[0:72, :]
    for i in range(12):
        v0 = c1_ref[2 * i].reshape(6, 24, BN)
        v1 = c1_ref[2 * i + 1].reshape(6, 24, BN)
        m = jnp.maximum(jnp.maximum(v0[:, 0:12, :], v0[:, 12:24, :]),
                        jnp.maximum(v1[:, 0:12, :], v1[:, 12:24, :]))
        p1_ref[i] = jnp.maximum(m.reshape(72, BN) + b1, 0.0).astype(bf16)

    for i in range(8):
        win = p1_ref[i:i + 5, :, :].reshape(5 * 72, BN)
        c2_ref[i] = jnp.dot(w2_ref[...], win, preferred_element_type=f32)

    b2 = bp_ref[72:136, :]
    for i in range(4):
        v0 = c2_ref[2 * i].reshape(16, 8, BN)
        v1 = c2_ref[2 * i + 1].reshape(16, 8, BN)
        m = jnp.maximum(jnp.maximum(v0[:, 0:4, :], v0[:, 4:8, :]),
                        jnp.maximum(v1[:, 0:4, :], v1[:, 4:8, :]))
        p2_ref[i] = jnp.maximum(m.reshape(64, BN) + b2, 0.0).astype(bf16)

    p2f = p2_ref[...].reshape(256, BN)
    h1 = jnp.dot(fc1_ref[...], p2f, preferred_element_type=f32)
    h1 = jnp.maximum(h1 + bp_ref[136:264, :], 0.0).astype(bf16)
    h2 = jax.lax.dot_general(f2_ref[...], h1, (((0,), (0,)), ((), ())),
                             preferred_element_type=f32)
    h2 = jnp.maximum(h2 + bp_ref[264:392, :], 0.0).astype(bf16)
    res = jax.lax.dot_general(h2, f3_ref[...], (((0,), (0,)), ((), ())),
                              preferred_element_type=f32)
    out_ref[...] = (res + bias_ref[4:5, :])[:, :16]


def _np_o1():
    o = np.zeros((5, 5, 2, 12, 5, 28), np.float32)
    for di in range(5):
        for dj in range(5):
            for p in range(2):
                for jj in range(12):
                    o[di, dj, p, jj, di, 2 * jj + p + dj] = 1.0
    return o.reshape(25, 2 * 12 * 5 * 28)


def _np_o2():
    oa = np.zeros((128, 2, 4, 5, 6, 12), np.float32)
    ob = np.zeros((128, 2, 4, 5, 6, 12), np.float32)
    for k in range(25):
        di, dj = divmod(k, 5)
        for ci in range(6):
            r = 6 * k + ci
            for p in range(2):
                for jj in range(4):
                    tgt = oa if k < 21 else ob
                    row = r if k < 21 else r - 126
                    tgt[row, p, jj, di, ci, 2 * jj + p + dj] = 1.0
    return oa.reshape(128, 2880), ob.reshape(128, 2880)


def _np_p1():
    pa = np.zeros((128, 256), np.float32)
    pb = np.zeros((128, 256), np.float32)
    for k in range(256):
        i2, rem = divmod(k, 64)
        c, j2 = divmod(rem, 4)
        pos = i2 * 4 + j2
        if pos < 8:
            pa[16 * pos + c, k] = 1.0
        else:
            pb[16 * (pos - 8) + c, k] = 1.0
    return pa, pb


_O1 = _np_o1()
_O2A, _O2B = _np_o2()
_P1A, _P1B = _np_p1()


def _pack(w1r, w2a, w2b, f1a, f1b, biases):
    f32 = jnp.float32
    bf16 = jnp.bfloat16
    dot0 = lambda a, b: jax.lax.dot_general(
        a.astype(f32), b, (((0,), (0,)), ((), ())), preferred_element_type=f32)

    w1row = dot0(w1r[:25, :6], _O1).reshape(6, 2, 12, 140)
    w1row = w1row.reshape(144, 140).astype(bf16)

    w2row = (dot0(w2a[:, :16], _O2A) + dot0(w2b[:, :16], _O2B))
    w2row = w2row.reshape(16, 2, 4, 360).reshape(128, 360).astype(bf16)

    fc1w = (dot0(f1a, _P1A) + dot0(f1b, _P1B)).astype(bf16)

    bpack = jnp.concatenate([jnp.repeat(biases[0, :6], 12),
                             jnp.repeat(biases[1, :16], 4),
                             biases[2], biases[3]])
    bpack = jnp.broadcast_to(bpack[:, None], (392, BN))
    return w1row, w2row, fc1w, bpack


def kernel(x_nchw, w1r, w2a, w2b, f1a, f1b, f2, f3, biases):
    n = x_nchw.shape[0]
    assert x_nchw.shape[1:] == (1, 28, 28), x_nchw.shape
    w1row, w2row, fc1w, bpack = _pack(w1r, w2a, w2b, f1a, f1b, biases)

    n_pad = ((n + BN - 1) // BN) * BN
    xt = jnp.transpose(x_nchw, (2, 3, 1, 0)).reshape(28 * 28, n)
    if n_pad != n:
        xt = jnp.pad(xt, ((0, 0), (0, n_pad - n)))

    const = lambda b: (0, 0)
    out = pl.pallas_call(
        _fused_kernel,
        out_shape=jax.ShapeDtypeStruct((n_pad, 16), jnp.float32),
        grid=(n_pad // BN,),
        in_specs=[
            pl.BlockSpec((28 * 28, BN), lambda b: (0, b)),
            pl.BlockSpec((144, 140), const),
            pl.BlockSpec((128, 360), const),
            pl.BlockSpec((128, 256), const),
            pl.BlockSpec((128, 128), const),
            pl.BlockSpec((128, 128), const),
            pl.BlockSpec((392, BN), const),
            pl.BlockSpec((5, 128), const),
        ],
        out_specs=pl.BlockSpec((BN, 16), lambda b: (b, 0)),
        scratch_shapes=[
            pltpu.VMEM((24, 144, BN), jnp.float32),
            pltpu.VMEM((12, 72, BN), jnp.bfloat16),
            pltpu.VMEM((8, 128, BN), jnp.float32),
            pltpu.VMEM((4, 64, BN), jnp.bfloat16),
        ],
        compiler_params=pltpu.CompilerParams(
            dimension_semantics=("parallel",),
            vmem_limit_bytes=64 * 1024 * 1024,
        ),
    )(xt, w1row, w2row, fc1w, f2.astype(jnp.bfloat16),
      f3.astype(jnp.bfloat16), bpack, biases)
    return out[:n, :10]


# speedup vs baseline: 1.6666x; 1.6666x over previous
import jax
import jax.numpy as jnp
import numpy as np
from jax.experimental import pallas as pl
from jax.experimental.pallas import tpu as pltpu

BN = 512


def _fused_kernel(x_ref, w1_ref, w2_ref, fc1_ref, f2_ref, f3_ref, bp_ref,
                  bias_ref, out_ref, xt_ref, c1_ref, p1_ref, c2_ref, p2_ref):
    f32 = jnp.float32
    bf16 = jnp.bfloat16

    for g in range(BN // 128):
        xt_ref[:, 128 * g:128 * (g + 1)] = x_ref[:, 0, g, :].astype(bf16)

    for i in range(24):
        c1_ref[i] = jnp.dot(w1_ref[...], xt_ref[pl.ds(28 * i, 140), :],
                            preferred_element_type=f32)

    b1 = bp_ref[0:72, :]
    for i in range(12):
        v0 = c1_ref[2 * i].reshape(6, 24, BN)
        v1 = c1_ref[2 * i + 1].reshape(6, 24, BN)
        m = jnp.maximum(jnp.maximum(v0[:, 0:12, :], v0[:, 12:24, :]),
                        jnp.maximum(v1[:, 0:12, :], v1[:, 12:24, :]))
        p1_ref[i] = jnp.maximum(m.reshape(72, BN) + b1, 0.0).astype(bf16)

    for i in range(8):
        win = p1_ref[i:i + 5, :, :].reshape(5 * 72, BN)
        c2_ref[i] = jnp.dot(w2_ref[...], win, preferred_element_type=f32)

    b2 = bp_ref[72:136, :]
    for i in range(4):
        v0 = c2_ref[2 * i].reshape(16, 8, BN)
        v1 = c2_ref[2 * i + 1].reshape(16, 8, BN)
        m = jnp.maximum(jnp.maximum(v0[:, 0:4, :], v0[:, 4:8, :]),
                        jnp.maximum(v1[:, 0:4, :], v1[:, 4:8, :]))
        p2_ref[i] = jnp.maximum(m.reshape(64, BN) + b2, 0.0).astype(bf16)

    p2f = p2_ref[...].reshape(256, BN)
    h1 = jnp.dot(fc1_ref[...], p2f, preferred_element_type=f32)
    h1 = jnp.maximum(h1 + bp_ref[136:264, :], 0.0).astype(bf16)
    h2 = jax.lax.dot_general(f2_ref[...], h1, (((0,), (0,)), ((), ())),
                             preferred_element_type=f32)
    h2 = jnp.maximum(h2 + bp_ref[264:392, :], 0.0).astype(bf16)
    res = jax.lax.dot_general(h2, f3_ref[...], (((0,), (0,)), ((), ())),
                              preferred_element_type=f32)
    out_ref[...] = (res + bias_ref[4:5, :])[:, :16]


def _np_o1():
    o = np.zeros((5, 5, 2, 12, 5, 28), np.float32)
    for di in range(5):
        for dj in range(5):
            for p in range(2):
                for jj in range(12):
                    o[di, dj, p, jj, di, 2 * jj + p + dj] = 1.0
    return o.reshape(25, 2 * 12 * 5 * 28)


def _np_o2():
    oa = np.zeros((128, 2, 4, 5, 6, 12), np.float32)
    ob = np.zeros((128, 2, 4, 5, 6, 12), np.float32)
    for k in range(25):
        di, dj = divmod(k, 5)
        for ci in range(6):
            r = 6 * k + ci
            for p in range(2):
                for jj in range(4):
                    tgt = oa if k < 21 else ob
                    row = r if k < 21 else r - 126
                    tgt[row, p, jj, di, ci, 2 * jj + p + dj] = 1.0
    return oa.reshape(128, 2880), ob.reshape(128, 2880)


def _np_p1():
    pa = np.zeros((128, 256), np.float32)
    pb = np.zeros((128, 256), np.float32)
    for k in range(256):
        i2, rem = divmod(k, 64)
        c, j2 = divmod(rem, 4)
        pos = i2 * 4 + j2
        if pos < 8:
            pa[16 * pos + c, k] = 1.0
        else:
            pb[16 * (pos - 8) + c, k] = 1.0
    return pa, pb


_O1 = _np_o1()
_O2A, _O2B = _np_o2()
_P1A, _P1B = _np_p1()


def _pack(w1r, w2a, w2b, f1a, f1b, biases):
    f32 = jnp.float32
    bf16 = jnp.bfloat16
    dot0 = lambda a, b: jax.lax.dot_general(
        a.astype(f32), b, (((0,), (0,)), ((), ())), preferred_element_type=f32)

    w1row = dot0(w1r[:25, :6], _O1).reshape(6, 2, 12, 140)
    w1row = w1row.reshape(144, 140).astype(bf16)

    w2row = (dot0(w2a[:, :16], _O2A) + dot0(w2b[:, :16], _O2B))
    w2row = w2row.reshape(16, 2, 4, 360).reshape(128, 360).astype(bf16)

    fc1w = (dot0(f1a, _P1A) + dot0(f1b, _P1B)).astype(bf16)

    bpack = jnp.concatenate([jnp.repeat(biases[0, :6], 12),
                             jnp.repeat(biases[1, :16], 4),
                             biases[2], biases[3]])
    bpack = jnp.broadcast_to(bpack[:, None], (392, BN))
    return w1row, w2row, fc1w, bpack


def kernel(x_nchw, w1r, w2a, w2b, f1a, f1b, f2, f3, biases):
    n = x_nchw.shape[0]
    assert x_nchw.shape[1:] == (1, 28, 28), x_nchw.shape
    w1row, w2row, fc1w, bpack = _pack(w1r, w2a, w2b, f1a, f1b, biases)

    n_pad = ((n + BN - 1) // BN) * BN
    if n_pad != n:
        x_nchw = jnp.pad(x_nchw, ((0, n_pad - n), (0, 0), (0, 0), (0, 0)))
    xv = jnp.transpose(x_nchw, (2, 3, 1, 0)).reshape(
        784, n_pad // BN, BN // 128, 128)

    const = lambda b: (0, 0)
    out = pl.pallas_call(
        _fused_kernel,
        out_shape=jax.ShapeDtypeStruct((n_pad, 16), jnp.float32),
        grid=(n_pad // BN,),
        in_specs=[
            pl.BlockSpec((28 * 28, 1, BN // 128, 128),
                         lambda b: (0, b, 0, 0)),
            pl.BlockSpec((144, 140), const),
            pl.BlockSpec((128, 360), const),
            pl.BlockSpec((128, 256), const),
            pl.BlockSpec((128, 128), const),
            pl.BlockSpec((128, 128), const),
            pl.BlockSpec((392, BN), const),
            pl.BlockSpec((5, 128), const),
        ],
        out_specs=pl.BlockSpec((BN, 16), lambda b: (b, 0)),
        scratch_shapes=[
            pltpu.VMEM((784, BN), jnp.bfloat16),
            pltpu.VMEM((24, 144, BN), jnp.float32),
            pltpu.VMEM((12, 72, BN), jnp.bfloat16),
            pltpu.VMEM((8, 128, BN), jnp.float32),
            pltpu.VMEM((4, 64, BN), jnp.bfloat16),
        ],
        compiler_params=pltpu.CompilerParams(
            dimension_semantics=("parallel",),
            vmem_limit_bytes=64 * 1024 * 1024,
        ),
    )(xv, w1row, w2row, fc1w, f2.astype(jnp.bfloat16),
      f3.astype(jnp.bfloat16), bpack, biases)
    return out[:n, :10]


# speedup vs baseline: 1.6888x; 1.0133x over previous
import jax
import jax.numpy as jnp
import numpy as np
from jax.experimental import pallas as pl
from jax.experimental.pallas import tpu as pltpu

BN = 1024


def _fused_kernel(x_ref, w1_ref, w2_ref, fc1_ref, f2_ref, f3_ref, bp_ref,
                  bias_ref, out_ref, xt_ref, c1_ref, p1_ref, c2_ref, p2_ref):
    f32 = jnp.float32
    bf16 = jnp.bfloat16

    for g in range(BN // 128):
        xt_ref[:, 128 * g:128 * (g + 1)] = x_ref[:, g, :].astype(bf16)

    for i in range(24):
        c1_ref[i] = jnp.dot(w1_ref[...], xt_ref[pl.ds(28 * i, 140), :],
                            preferred_element_type=f32)

    b1 = bp_ref[0:72, :]
    for i in range(12):
        v0 = c1_ref[2 * i].reshape(6, 24, BN)
        v1 = c1_ref[2 * i + 1].reshape(6, 24, BN)
        m = jnp.maximum(jnp.maximum(v0[:, 0:12, :], v0[:, 12:24, :]),
                        jnp.maximum(v1[:, 0:12, :], v1[:, 12:24, :]))
        p1_ref[i] = jnp.maximum(m.reshape(72, BN) + b1, 0.0).astype(bf16)

    for i in range(8):
        win = p1_ref[i:i + 5, :, :].reshape(5 * 72, BN)
        c2_ref[i] = jnp.dot(w2_ref[...], win, preferred_element_type=f32)

    b2 = bp_ref[72:136, :]
    for i in range(4):
        v0 = c2_ref[2 * i].reshape(16, 8, BN)
        v1 = c2_ref[2 * i + 1].reshape(16, 8, BN)
        m = jnp.maximum(jnp.maximum(v0[:, 0:4, :], v0[:, 4:8, :]),
                        jnp.maximum(v1[:, 0:4, :], v1[:, 4:8, :]))
        p2_ref[i] = jnp.maximum(m.reshape(64, BN) + b2, 0.0).astype(bf16)

    p2f = p2_ref[...].reshape(256, BN)
    h1 = jnp.dot(fc1_ref[...], p2f, preferred_element_type=f32)
    h1 = jnp.maximum(h1 + bp_ref[136:264, :], 0.0).astype(bf16)
    h2 = jax.lax.dot_general(f2_ref[...], h1, (((0,), (0,)), ((), ())),
                             preferred_element_type=f32)
    h2 = jnp.maximum(h2 + bp_ref[264:392, :], 0.0).astype(bf16)
    res = jax.lax.dot_general(h2, f3_ref[...], (((0,), (0,)), ((), ())),
                              preferred_element_type=f32)
    out_ref[...] = (res + bias_ref[4:5, :])[:, :16]


def _np_o1():
    o = np.zeros((5, 5, 2, 12, 5, 28), np.float32)
    for di in range(5):
        for dj in range(5):
            for p in range(2):
                for jj in range(12):
                    o[di, dj, p, jj, di, 2 * jj + p + dj] = 1.0
    return o.reshape(25, 2 * 12 * 5 * 28)


def _np_o2():
    oa = np.zeros((128, 2, 4, 5, 6, 12), np.float32)
    ob = np.zeros((128, 2, 4, 5, 6, 12), np.float32)
    for k in range(25):
        di, dj = divmod(k, 5)
        for ci in range(6):
            r = 6 * k + ci
            for p in range(2):
                for jj in range(4):
                    tgt = oa if k < 21 else ob
                    row = r if k < 21 else r - 126
                    tgt[row, p, jj, di, ci, 2 * jj + p + dj] = 1.0
    return oa.reshape(128, 2880), ob.reshape(128, 2880)


def _np_p1():
    pa = np.zeros((128, 256), np.float32)
    pb = np.zeros((128, 256), np.float32)
    for k in range(256):
        i2, rem = divmod(k, 64)
        c, j2 = divmod(rem, 4)
        pos = i2 * 4 + j2
        if pos < 8:
            pa[16 * pos + c, k] = 1.0
        else:
            pb[16 * (pos - 8) + c, k] = 1.0
    return pa, pb


_O1 = _np_o1()
_O2A, _O2B = _np_o2()
_P1A, _P1B = _np_p1()


def _pack(w1r, w2a, w2b, f1a, f1b, biases):
    f32 = jnp.float32
    bf16 = jnp.bfloat16
    dot0 = lambda a, b: jax.lax.dot_general(
        a.astype(f32), b, (((0,), (0,)), ((), ())), preferred_element_type=f32)

    w1row = dot0(w1r[:25, :6], _O1).reshape(6, 2, 12, 140)
    w1row = w1row.reshape(144, 140).astype(bf16)

    w2row = (dot0(w2a[:, :16], _O2A) + dot0(w2b[:, :16], _O2B))
    w2row = w2row.reshape(16, 2, 4, 360).reshape(128, 360).astype(bf16)

    fc1w = (dot0(f1a, _P1A) + dot0(f1b, _P1B)).astype(bf16)

    bpack = jnp.concatenate([jnp.repeat(biases[0, :6], 12),
                             jnp.repeat(biases[1, :16], 4),
                             biases[2], biases[3]])
    bpack = jnp.broadcast_to(bpack[:, None], (392, BN))
    return w1row, w2row, fc1w, bpack


def kernel(x_nchw, w1r, w2a, w2b, f1a, f1b, f2, f3, biases):
    n = x_nchw.shape[0]
    assert x_nchw.shape[1:] == (1, 28, 28), x_nchw.shape
    w1row, w2row, fc1w, bpack = _pack(w1r, w2a, w2b, f1a, f1b, biases)

    n_pad = ((n + BN - 1) // BN) * BN
    if n_pad != n:
        x_nchw = jnp.pad(x_nchw, ((0, n_pad - n), (0, 0), (0, 0), (0, 0)))
    xv = jnp.transpose(x_nchw, (2, 3, 1, 0)).reshape(784, n_pad // 128, 128)

    const = lambda b: (0, 0)
    out = pl.pallas_call(
        _fused_kernel,
        out_shape=jax.ShapeDtypeStruct((n_pad, 16), jnp.float32),
        grid=(n_pad // BN,),
        in_specs=[
            pl.BlockSpec((28 * 28, BN // 128, 128),
                         lambda b: (0, b, 0)),
            pl.BlockSpec((144, 140), const),
            pl.BlockSpec((128, 360), const),
            pl.BlockSpec((128, 256), const),
            pl.BlockSpec((128, 128), const),
            pl.BlockSpec((128, 128), const),
            pl.BlockSpec((392, BN), const),
            pl.BlockSpec((5, 128), const),
        ],
        out_specs=pl.BlockSpec((BN, 16), lambda b: (b, 0)),
        scratch_shapes=[
            pltpu.VMEM((784, BN), jnp.bfloat16),
            pltpu.VMEM((24, 144, BN), jnp.float32),
            pltpu.VMEM((12, 72, BN), jnp.bfloat16),
            pltpu.VMEM((8, 128, BN), jnp.float32),
            pltpu.VMEM((4, 64, BN), jnp.bfloat16),
        ],
        compiler_params=pltpu.CompilerParams(
            dimension_semantics=("parallel",),
            vmem_limit_bytes=64 * 1024 * 1024,
        ),
    )(xv, w1row, w2row, fc1w, f2.astype(jnp.bfloat16),
      f3.astype(jnp.bfloat16), bpack, biases)
    return out[:n, :10]


# speedup vs baseline: 2.0075x; 1.1887x over previous
import jax
import jax.numpy as jnp
import numpy as np
from jax.experimental import pallas as pl
from jax.experimental.pallas import tpu as pltpu

BN = 1024


def _fused_kernel(x_ref, w1_ref, w2_ref, fc1_ref, f2_ref, f3_ref, bp_ref,
                  bias_ref, out_ref, xt_ref, c1_ref, p1_ref, c2_ref, p2_ref):
    f32 = jnp.float32
    bf16 = jnp.bfloat16

    xc = x_ref[...].astype(bf16)
    for g in range(BN // 128):
        xt_ref[:, 128 * g:128 * (g + 1)] = xc[:, g, :]

    for i in range(24):
        c1_ref[i] = jnp.dot(w1_ref[...], xt_ref[pl.ds(28 * i, 140), :],
                            preferred_element_type=f32)

    b1 = bp_ref[0:72, :]
    for i in range(12):
        v0 = c1_ref[2 * i].reshape(6, 24, BN)
        v1 = c1_ref[2 * i + 1].reshape(6, 24, BN)
        m = jnp.maximum(jnp.maximum(v0[:, 0:12, :], v0[:, 12:24, :]),
                        jnp.maximum(v1[:, 0:12, :], v1[:, 12:24, :]))
        p1_ref[i] = jnp.maximum(m.reshape(72, BN) + b1, 0.0).astype(bf16)

    for i in range(8):
        win = p1_ref[i:i + 5, :, :].reshape(5 * 72, BN)
        c2_ref[i] = jnp.dot(w2_ref[...], win, preferred_element_type=f32)

    b2 = bp_ref[72:136, :]
    for i in range(4):
        v0 = c2_ref[2 * i].reshape(16, 8, BN)
        v1 = c2_ref[2 * i + 1].reshape(16, 8, BN)
        m = jnp.maximum(jnp.maximum(v0[:, 0:4, :], v0[:, 4:8, :]),
                        jnp.maximum(v1[:, 0:4, :], v1[:, 4:8, :]))
        p2_ref[i] = jnp.maximum(m.reshape(64, BN) + b2, 0.0).astype(bf16)

    p2f = p2_ref[...].reshape(256, BN)
    h1 = jnp.dot(fc1_ref[...], p2f, preferred_element_type=f32)
    h1 = jnp.maximum(h1 + bp_ref[136:264, :], 0.0).astype(bf16)
    h2 = jax.lax.dot_general(f2_ref[...], h1, (((0,), (0,)), ((), ())),
                             preferred_element_type=f32)
    h2 = jnp.maximum(h2 + bp_ref[264:392, :], 0.0).astype(bf16)
    res = jax.lax.dot_general(h2, f3_ref[...], (((0,), (0,)), ((), ())),
                              preferred_element_type=f32)
    out_ref[...] = (res + bias_ref[4:5, :])[:, :16]


def _np_o1():
    o = np.zeros((5, 5, 2, 12, 5, 28), np.float32)
    for di in range(5):
        for dj in range(5):
            for p in range(2):
                for jj in range(12):
                    o[di, dj, p, jj, di, 2 * jj + p + dj] = 1.0
    return o.reshape(25, 2 * 12 * 5 * 28)


def _np_o2():
    oa = np.zeros((128, 2, 4, 5, 6, 12), np.float32)
    ob = np.zeros((128, 2, 4, 5, 6, 12), np.float32)
    for k in range(25):
        di, dj = divmod(k, 5)
        for ci in range(6):
            r = 6 * k + ci
            for p in range(2):
                for jj in range(4):
                    tgt = oa if k < 21 else ob
                    row = r if k < 21 else r - 126
                    tgt[row, p, jj, di, ci, 2 * jj + p + dj] = 1.0
    return oa.reshape(128, 2880), ob.reshape(128, 2880)


def _np_p1():
    pa = np.zeros((128, 256), np.float32)
    pb = np.zeros((128, 256), np.float32)
    for k in range(256):
        i2, rem = divmod(k, 64)
        c, j2 = divmod(rem, 4)
        pos = i2 * 4 + j2
        if pos < 8:
            pa[16 * pos + c, k] = 1.0
        else:
            pb[16 * (pos - 8) + c, k] = 1.0
    return pa, pb


_O1 = _np_o1()
_O2A, _O2B = _np_o2()
_P1A, _P1B = _np_p1()


def _pack(w1r, w2a, w2b, f1a, f1b, biases):
    f32 = jnp.float32
    bf16 = jnp.bfloat16
    dot0 = lambda a, b: jax.lax.dot_general(
        a.astype(f32), b, (((0,), (0,)), ((), ())), preferred_element_type=f32)

    w1row = dot0(w1r[:25, :6], _O1).reshape(6, 2, 12, 140)
    w1row = w1row.reshape(144, 140).astype(bf16)

    w2row = (dot0(w2a[:, :16], _O2A) + dot0(w2b[:, :16], _O2B))
    w2row = w2row.reshape(16, 2, 4, 360).reshape(128, 360).astype(bf16)

    fc1w = (dot0(f1a, _P1A) + dot0(f1b, _P1B)).astype(bf16)

    bpack = jnp.concatenate([jnp.repeat(biases[0, :6], 12),
                             jnp.repeat(biases[1, :16], 4),
                             biases[2], biases[3]])
    bpack = jnp.broadcast_to(bpack[:, None], (392, BN))
    return w1row, w2row, fc1w, bpack


def kernel(x_nchw, w1r, w2a, w2b, f1a, f1b, f2, f3, biases):
    n = x_nchw.shape[0]
    assert x_nchw.shape[1:] == (1, 28, 28), x_nchw.shape
    w1row, w2row, fc1w, bpack = _pack(w1r, w2a, w2b, f1a, f1b, biases)

    n_pad = ((n + BN - 1) // BN) * BN
    if n_pad != n:
        x_nchw = jnp.pad(x_nchw, ((0, n_pad - n), (0, 0), (0, 0), (0, 0)))
    xv = jnp.transpose(x_nchw, (2, 3, 1, 0)).reshape(784, n_pad // 128, 128)

    const = lambda b: (0, 0)
    out = pl.pallas_call(
        _fused_kernel,
        out_shape=jax.ShapeDtypeStruct((n_pad, 16), jnp.float32),
        grid=(n_pad // BN,),
        in_specs=[
            pl.BlockSpec((28 * 28, BN // 128, 128),
                         lambda b: (0, b, 0)),
            pl.BlockSpec((144, 140), const),
            pl.BlockSpec((128, 360), const),
            pl.BlockSpec((128, 256), const),
            pl.BlockSpec((128, 128), const),
            pl.BlockSpec((128, 128), const),
            pl.BlockSpec((392, BN), const),
            pl.BlockSpec((5, 128), const),
        ],
        out_specs=pl.BlockSpec((BN, 16), lambda b: (b, 0)),
        scratch_shapes=[
            pltpu.VMEM((784, BN), jnp.bfloat16),
            pltpu.VMEM((24, 144, BN), jnp.float32),
            pltpu.VMEM((12, 72, BN), jnp.bfloat16),
            pltpu.VMEM((8, 128, BN), jnp.float32),
            pltpu.VMEM((4, 64, BN), jnp.bfloat16),
        ],
        compiler_params=pltpu.CompilerParams(
            dimension_semantics=("parallel",),
            vmem_limit_bytes=64 * 1024 * 1024,
        ),
    )(xv, w1row, w2row, fc1w, f2.astype(jnp.bfloat16),
      f3.astype(jnp.bfloat16), bpack, biases)
    return out[:n, :10]
